# baseline (speedup 1.0000x reference)
import functools

import jax
import jax.numpy as jnp
from jax import lax
from jax.experimental import pallas as pl
from jax.experimental.pallas import tpu as pltpu
from jax.experimental.pallas import tpu_sc as plsc

D = 128
BATCH = 4
SEQ = 2048
N_TOK = BATCH * SEQ
NC = 2
NS = 16
NW = NC * NS
PER_W = N_TOK // NW
W_PER_ROW = SEQ // PER_W

_mesh = plsc.VectorSubcoreMesh(core_axis_name="c", subcore_axis_name="s")


@functools.partial(
    pl.kernel,
    out_type=jax.ShapeDtypeStruct((N_TOK, D), jnp.float32),
    mesh=_mesh,
    scratch_types=[
        pltpu.VMEM((PER_W,), jnp.int32),
        pltpu.VMEM((PER_W,), jnp.int32),
        pltpu.VMEM((PER_W, D), jnp.float32),
        pltpu.VMEM_SHARED((SEQ, D), jnp.float32),
        pltpu.SemaphoreType.DMA,
        pltpu.SemaphoreType.DMA,
        pltpu.SemaphoreType.DMA,
        pltpu.SemaphoreType.DMA,
    ],
)
def _embed(ids_hbm, pos_hbm, wte_hbm, wpe_hbm, out_hbm,
           ti_v, pi_v, a, wpe_sh, si0, si1, sg, sp):
    wid = lax.axis_index("s") * NC + lax.axis_index("c")
    brow = wid // W_PER_ROW
    s0 = (wid % W_PER_ROW) * PER_W
    ci0 = pltpu.async_copy(ids_hbm.at[brow, pl.ds(s0, PER_W)], ti_v, si0)
    ci1 = pltpu.async_copy(pos_hbm.at[brow, pl.ds(s0, PER_W)], pi_v, si1)
    @pl.when(lax.axis_index("s") == 0)
    def _():
        pltpu.async_copy(wpe_hbm, wpe_sh, sp).wait()

    ci0.wait()
    ga = pltpu.async_copy(wte_hbm.at[ti_v], a, sg)
    plsc.subcore_barrier()
    ci1.wait()
    ga.wait()
    gb = pltpu.async_copy(wpe_sh.at[pi_v], a, sg, add=True)
    gb.wait()
    co = pltpu.async_copy(a, out_hbm.at[pl.ds(wid * PER_W, PER_W)], sg)
    co.wait()


def kernel(input_ids, position_ids, wte, wpe):
    out = _embed(input_ids.astype(jnp.int32), position_ids.astype(jnp.int32),
                 wte, wpe)
    return out.reshape(input_ids.shape + (wte.shape[1],))
